# baseline (speedup 1.0000x reference)
import jax
import jax.numpy as jnp
from jax.experimental import pallas as pl

_CHUNK = 128


def _add_kernel(x_ref, p_ref, o_ref):
    o_ref[...] = x_ref[...] + p_ref[...][None, :, :]


def kernel(inputs, pos_table):
    b, s, d = inputs.shape
    chunk = min(_CHUNK, s)
    return pl.pallas_call(
        _add_kernel,
        grid=(s // chunk,),
        in_specs=[
            pl.BlockSpec((b, chunk, d), lambda i: (0, i, 0)),
            pl.BlockSpec((chunk, d), lambda i: (i, 0)),
        ],
        out_specs=pl.BlockSpec((b, chunk, d), lambda i: (0, i, 0)),
        out_shape=jax.ShapeDtypeStruct((b, s, d), inputs.dtype),
    )(inputs, pos_table)


# speedup vs baseline: 1.0679x; 1.0679x over previous
import functools

import jax
import jax.numpy as jnp
from jax import lax
from jax.experimental import pallas as pl
from jax.experimental.pallas import tpu as pltpu
from jax.experimental.pallas import tpu_sc as plsc

_CHUNK = 512


def _add_kernel(x_ref, p_ref, o_ref):
    o_ref[...] = x_ref[...] + p_ref[...][None, :, :]


def _kernel_tc(inputs, pos_table):
    b, s, d = inputs.shape
    chunk = min(_CHUNK, s)
    return pl.pallas_call(
        _add_kernel,
        grid=(s // chunk,),
        in_specs=[
            pl.BlockSpec((b, chunk, d), lambda i: (0, i, 0)),
            pl.BlockSpec((chunk, d), lambda i: (i, 0)),
        ],
        out_specs=pl.BlockSpec((b, chunk, d), lambda i: (0, i, 0)),
        out_shape=jax.ShapeDtypeStruct((b, s, d), inputs.dtype),
    )(inputs, pos_table)


_NC = 2
_NS = 16
_NW = _NC * _NS
_CH_SC = 64


def _kernel_sc(inputs, pos_table):
    b, s, d = inputs.shape
    spw = s // _NW
    mesh = plsc.VectorSubcoreMesh(
        core_axis_name="c", subcore_axis_name="s", num_cores=_NC, num_subcores=_NS
    )

    @functools.partial(
        pl.kernel,
        mesh=mesh,
        out_type=jax.ShapeDtypeStruct((b * s, d), inputs.dtype),
        scratch_types=[
            pltpu.VMEM((_CH_SC,), jnp.int32),
            pltpu.VMEM((_CH_SC, d), jnp.float32),
            pltpu.SemaphoreType.DMA,
        ],
    )
    def k(in_hbm, tab_hbm, out_hbm, idx_v, x_v, sem):
        wid = lax.axis_index("s") * _NC + lax.axis_index("c")
        s_base = wid * spw
        for c in range(spw // _CH_SC):
            s0 = s_base + c * _CH_SC
            for j in range(_CH_SC // 16):
                idx_v[pl.ds(j * 16, 16)] = s0 + j * 16 + lax.iota(jnp.int32, 16)
            for bi in range(b):
                r0 = bi * s + s0
                pltpu.sync_copy(in_hbm.at[pl.ds(r0, _CH_SC)], x_v)
                pltpu.async_copy(tab_hbm.at[idx_v], x_v, sem, add=True).wait()
                pltpu.sync_copy(x_v, out_hbm.at[pl.ds(r0, _CH_SC)])

    out = k(inputs.reshape(b * s, d), pos_table)
    return out.reshape(b, s, d)


kernel = _kernel_tc


# speedup vs baseline: 1.0769x; 1.0084x over previous
import functools

import jax
import jax.numpy as jnp
from jax import lax
from jax.experimental import pallas as pl
from jax.experimental.pallas import tpu as pltpu
from jax.experimental.pallas import tpu_sc as plsc

_CHUNK = 512


def _add_kernel(x_ref, p_ref, o_ref):
    o_ref[...] = x_ref[...] + p_ref[...][None, :, :]


def _kernel_tc(inputs, pos_table):
    b, s, d = inputs.shape
    chunk = min(_CHUNK, s)
    return pl.pallas_call(
        _add_kernel,
        grid=(s // chunk,),
        in_specs=[
            pl.BlockSpec((b, chunk, d), lambda i: (0, i, 0)),
            pl.BlockSpec((chunk, d), lambda i: (i, 0)),
        ],
        out_specs=pl.BlockSpec((b, chunk, d), lambda i: (0, i, 0)),
        out_shape=jax.ShapeDtypeStruct((b, s, d), inputs.dtype),
    )(inputs, pos_table)


def _add_kernel2(x_ref, p_ref, o_ref):
    o_ref[...] = x_ref[...] + p_ref[...][None, :, :]


def _kernel_tc2(inputs, pos_table, bb=1, chunk=2048):
    b, s, d = inputs.shape
    return pl.pallas_call(
        _add_kernel2,
        grid=(s // chunk, b // bb),
        in_specs=[
            pl.BlockSpec((bb, chunk, d), lambda i, j: (j, i, 0)),
            pl.BlockSpec((chunk, d), lambda i, j: (i, 0)),
        ],
        out_specs=pl.BlockSpec((bb, chunk, d), lambda i, j: (j, i, 0)),
        out_shape=jax.ShapeDtypeStruct((b, s, d), inputs.dtype),
    )(inputs, pos_table)


_NC = 2
_NS = 16
_NW = _NC * _NS
_CH_SC = 64


def _kernel_sc(inputs, pos_table):
    b, s, d = inputs.shape
    spw = s // _NW
    mesh = plsc.VectorSubcoreMesh(
        core_axis_name="c", subcore_axis_name="s", num_cores=_NC, num_subcores=_NS
    )

    @functools.partial(
        pl.kernel,
        mesh=mesh,
        out_type=jax.ShapeDtypeStruct((b * s, d), inputs.dtype),
        scratch_types=[
            pltpu.VMEM((_CH_SC,), jnp.int32),
            pltpu.VMEM((_CH_SC, d), jnp.float32),
            pltpu.SemaphoreType.DMA,
        ],
    )
    def k(in_hbm, tab_hbm, out_hbm, idx_v, x_v, sem):
        wid = lax.axis_index("s") * _NC + lax.axis_index("c")
        s_base = wid * spw
        for c in range(spw // _CH_SC):
            s0 = s_base + c * _CH_SC
            for j in range(_CH_SC // 16):
                idx_v[pl.ds(j * 16, 16)] = s0 + j * 16 + lax.iota(jnp.int32, 16)
            for bi in range(b):
                r0 = bi * s + s0
                pltpu.sync_copy(in_hbm.at[pl.ds(r0, _CH_SC)], x_v)
                pltpu.async_copy(tab_hbm.at[idx_v], x_v, sem, add=True).wait()
                pltpu.sync_copy(x_v, out_hbm.at[pl.ds(r0, _CH_SC)])

    out = k(inputs.reshape(b * s, d), pos_table)
    return out.reshape(b, s, d)


kernel = _kernel_tc2
